# baseline (speedup 1.0000x reference)
import functools

import jax
import jax.numpy as jnp
from jax.experimental import pallas as pl
from jax.experimental.pallas import tpu as pltpu

_B, _S, _D = 4, 2048, 4096
_E = 64
_K = 8
_ALPHA = 0.01
_T = _B * _S
_BT = 1024
_MINS = -(2 ** 31)
_SMASK = 0x7FFFFFFF


def _flip(b):
    return b ^ ((b >> 31) & _SMASK)


def _router_block(x_ref, wt_ref, b_ref, w_ref, id_ref, aux_ref,
                  psum_ref, cnt_ref):
    step = pl.program_id(0)
    nsteps = pl.num_programs(0)

    @pl.when(step == 0)
    def _init():
        psum_ref[...] = jnp.zeros_like(psum_ref)
        cnt_ref[...] = jnp.zeros_like(cnt_ref)

    x = x_ref[...]
    logits = jnp.dot(x, wt_ref[...], preferred_element_type=jnp.float32)
    logits = logits + b_ref[...]

    bits = jax.lax.bitcast_convert_type(logits, jnp.int32)
    iota = jax.lax.broadcasted_iota(jnp.int32, logits.shape, 1)
    key = (_flip(bits) & (-64)) | (63 - iota)

    work = key
    cols = []
    for _ in range(_K):
        mx = jnp.max(work, axis=-1, keepdims=True)
        work = jnp.where(work == mx, _MINS, work)
        cols.append(mx)

    packed = jnp.concatenate(cols, axis=1)
    imat = 63 - (packed & 63)
    lsel = jax.lax.bitcast_convert_type(
        _flip(packed & (-64)), jnp.float32)
    m = lsel[:, 0:1]
    wexp = jnp.exp(lsel - m)
    w_ref[...] = wexp * (1.0 / jnp.sum(wexp, axis=1, keepdims=True))
    id_ref[...] = imat

    e = jnp.exp(logits - m)
    probs = e * (1.0 / jnp.sum(e, axis=-1, keepdims=True))
    sel = jnp.where(work == _MINS, 1.0, 0.0)
    psum_ref[...] += jnp.sum(probs, axis=0, keepdims=True)
    cnt_ref[...] += jnp.sum(sel, axis=0, keepdims=True)

    @pl.when(step == nsteps - 1)
    def _finish():
        scale = _ALPHA * _E / (float(_T) * _K * float(_T))
        aux = jnp.sum(psum_ref[...] * cnt_ref[...], keepdims=True) * scale
        aux_ref[...] = aux.reshape(1, 1)


@functools.partial(jax.jit, static_argnames=("interpret",))
def kernel(x, W, b, interpret=False):
    xt = x.reshape(_T, _D)
    wt = W.T
    b2 = b.reshape(1, _E)
    grid = (_T // _BT,)
    w_out, id_out, aux = pl.pallas_call(
        _router_block,
        grid=grid,
        in_specs=[
            pl.BlockSpec((_BT, _D), lambda i: (i, 0)),
            pl.BlockSpec((_D, _E), lambda i: (0, 0)),
            pl.BlockSpec((1, _E), lambda i: (0, 0)),
        ],
        out_specs=[
            pl.BlockSpec((_BT, _K), lambda i: (i, 0)),
            pl.BlockSpec((_BT, _K), lambda i: (i, 0)),
            pl.BlockSpec((1, 1), lambda i: (0, 0)),
        ],
        out_shape=[
            jax.ShapeDtypeStruct((_T, _K), jnp.float32),
            jax.ShapeDtypeStruct((_T, _K), jnp.int32),
            jax.ShapeDtypeStruct((1, 1), jnp.float32),
        ],
        scratch_shapes=[
            pltpu.VMEM((1, _E), jnp.float32),
            pltpu.VMEM((1, _E), jnp.float32),
        ],
        interpret=interpret,
    )(xt, wt, b2)
    return w_out, id_out, aux[0, 0]


# speedup vs baseline: 1.1227x; 1.1227x over previous
import functools

import jax
import jax.numpy as jnp
from jax.experimental import pallas as pl
from jax.experimental.pallas import tpu as pltpu

_B, _S, _D = 4, 2048, 4096
_E = 64
_K = 8
_ALPHA = 0.01
_T = _B * _S
_BT = 1024


def _router_block(x_ref, wt_ref, b_ref, w_ref, id_ref, aux_ref,
                  psum_ref, cnt_ref):
    step = pl.program_id(0)
    nsteps = pl.num_programs(0)

    @pl.when(step == 0)
    def _init():
        psum_ref[...] = jnp.zeros_like(psum_ref)
        cnt_ref[...] = jnp.zeros_like(cnt_ref)

    x = x_ref[...]
    logits = jnp.dot(x, wt_ref[...], preferred_element_type=jnp.float32)
    logits = logits + b_ref[...]

    m = jnp.max(logits, axis=-1, keepdims=True)
    e = jnp.exp(logits - m)
    probs = e * (1.0 / jnp.sum(e, axis=-1, keepdims=True))

    iota = jax.lax.broadcasted_iota(jnp.int32, e.shape, 1)
    cif = (63 - iota).astype(jnp.float32)
    work = e
    selm = jnp.zeros(e.shape, dtype=jnp.float32)
    w_cols = []
    i_cols = []
    for _ in range(_K):
        mx = jnp.max(work, axis=-1, keepdims=True)
        hitm = work == mx
        pk = jnp.max(jnp.where(hitm, cif, -1.0), axis=-1,
                     keepdims=True)
        hit1 = jnp.logical_and(hitm, cif == pk)
        selm = jnp.where(hit1, 1.0, selm)
        work = jnp.where(hit1, -1.0, work)
        w_cols.append(mx)
        i_cols.append(pk)

    wmat = jnp.concatenate(w_cols, axis=1)
    imat = (63.0 - jnp.concatenate(i_cols, axis=1)).astype(jnp.int32)
    w_ref[...] = wmat * (1.0 / jnp.sum(wmat, axis=1, keepdims=True))
    id_ref[...] = imat

    psum_ref[...] += jnp.sum(probs, axis=0, keepdims=True)
    cnt_ref[...] += jnp.sum(selm, axis=0, keepdims=True)

    @pl.when(step == nsteps - 1)
    def _finish():
        scale = _ALPHA * _E / (float(_T) * _K * float(_T))
        aux = jnp.sum(psum_ref[...] * cnt_ref[...], keepdims=True) * scale
        aux_ref[...] = aux.reshape(1, 1)


@functools.partial(jax.jit, static_argnames=("interpret",))
def kernel(x, W, b, interpret=False):
    xt = x.reshape(_T, _D)
    wt = W.T
    b2 = b.reshape(1, _E)
    grid = (_T // _BT,)
    w_out, id_out, aux = pl.pallas_call(
        _router_block,
        grid=grid,
        in_specs=[
            pl.BlockSpec((_BT, _D), lambda i: (i, 0)),
            pl.BlockSpec((_D, _E), lambda i: (0, 0)),
            pl.BlockSpec((1, _E), lambda i: (0, 0)),
        ],
        out_specs=[
            pl.BlockSpec((_BT, _K), lambda i: (i, 0)),
            pl.BlockSpec((_BT, _K), lambda i: (i, 0)),
            pl.BlockSpec((1, 1), lambda i: (0, 0)),
        ],
        out_shape=[
            jax.ShapeDtypeStruct((_T, _K), jnp.float32),
            jax.ShapeDtypeStruct((_T, _K), jnp.int32),
            jax.ShapeDtypeStruct((1, 1), jnp.float32),
        ],
        scratch_shapes=[
            pltpu.VMEM((1, _E), jnp.float32),
            pltpu.VMEM((1, _E), jnp.float32),
        ],
        interpret=interpret,
    )(xt, wt, b2)
    return w_out, id_out, aux[0, 0]


# speedup vs baseline: 1.1463x; 1.0210x over previous
import functools

import jax
import jax.numpy as jnp
from jax.experimental import pallas as pl
from jax.experimental.pallas import tpu as pltpu

_B, _S, _D = 4, 2048, 4096
_E = 64
_K = 8
_ALPHA = 0.01
_T = _B * _S
_BT = 1024


def _router_block(x_ref, wt_ref, b_ref, w_ref, id_ref, aux_ref,
                  psum_ref, cnt_ref):
    step = pl.program_id(0)
    nsteps = pl.num_programs(0)

    @pl.when(step == 0)
    def _init():
        psum_ref[...] = jnp.zeros_like(psum_ref)
        cnt_ref[...] = jnp.zeros_like(cnt_ref)

    x = x_ref[...]
    logits = jnp.dot(x, wt_ref[...], preferred_element_type=jnp.float32)
    logits = logits + b_ref[...]

    m = jnp.max(logits, axis=-1, keepdims=True)
    e = jnp.exp(logits - m)
    probs = e * (1.0 / jnp.sum(e, axis=-1, keepdims=True))

    iota = jax.lax.broadcasted_iota(jnp.int32, e.shape, 1)
    cif = (63 - iota).astype(jnp.float32)
    work = e
    w_cols = []
    i_cols = []
    for _ in range(_K):
        mx = jnp.max(work, axis=-1, keepdims=True)
        pk = jnp.max(jnp.where(work == mx, cif, -1.0), axis=-1,
                     keepdims=True)
        work = jnp.where(cif == pk, -1.0, work)
        w_cols.append(mx)
        i_cols.append(pk)

    wmat = jnp.concatenate(w_cols, axis=1)
    imat = (63.0 - jnp.concatenate(i_cols, axis=1)).astype(jnp.int32)
    w_ref[...] = wmat * (1.0 / jnp.sum(wmat, axis=1, keepdims=True))
    id_ref[...] = imat

    selm = jnp.where(work < 0.0, 1.0, 0.0)
    psum_ref[...] += jnp.sum(probs, axis=0, keepdims=True)
    cnt_ref[...] += jnp.sum(selm, axis=0, keepdims=True)

    @pl.when(step == nsteps - 1)
    def _finish():
        scale = _ALPHA * _E / (float(_T) * _K * float(_T))
        aux = jnp.sum(psum_ref[...] * cnt_ref[...], keepdims=True) * scale
        aux_ref[...] = aux.reshape(1, 1)


@functools.partial(jax.jit, static_argnames=("interpret",))
def kernel(x, W, b, interpret=False):
    xt = x.reshape(_T, _D)
    wt = W.T
    b2 = b.reshape(1, _E)
    grid = (_T // _BT,)
    w_out, id_out, aux = pl.pallas_call(
        _router_block,
        grid=grid,
        in_specs=[
            pl.BlockSpec((_BT, _D), lambda i: (i, 0)),
            pl.BlockSpec((_D, _E), lambda i: (0, 0)),
            pl.BlockSpec((1, _E), lambda i: (0, 0)),
        ],
        out_specs=[
            pl.BlockSpec((_BT, _K), lambda i: (i, 0)),
            pl.BlockSpec((_BT, _K), lambda i: (i, 0)),
            pl.BlockSpec((1, 1), lambda i: (0, 0)),
        ],
        out_shape=[
            jax.ShapeDtypeStruct((_T, _K), jnp.float32),
            jax.ShapeDtypeStruct((_T, _K), jnp.int32),
            jax.ShapeDtypeStruct((1, 1), jnp.float32),
        ],
        scratch_shapes=[
            pltpu.VMEM((1, _E), jnp.float32),
            pltpu.VMEM((1, _E), jnp.float32),
        ],
        interpret=interpret,
    )(xt, wt, b2)
    return w_out, id_out, aux[0, 0]


# speedup vs baseline: 1.1516x; 1.0046x over previous
import functools

import jax
import jax.numpy as jnp
from jax.experimental import pallas as pl
from jax.experimental.pallas import tpu as pltpu

_B, _S, _D = 4, 2048, 4096
_E = 64
_K = 8
_ALPHA = 0.01
_T = _B * _S
_BT = 1024
_HT = _BT // 2


def _router_block(x_ref, wt_ref, b_ref, w_ref, id_ref, aux_ref,
                  psum_ref, cnt_ref):
    step = pl.program_id(0)
    nsteps = pl.num_programs(0)

    @pl.when(step == 0)
    def _init():
        psum_ref[...] = jnp.zeros_like(psum_ref)
        cnt_ref[...] = jnp.zeros_like(cnt_ref)

    wt = wt_ref[...]
    bias = b_ref[...]
    neg = jnp.float32(-jnp.inf)

    for h in range(2):
        rows = pl.ds(h * _HT, _HT)
        logits = jnp.dot(x_ref[rows, :], wt,
                         preferred_element_type=jnp.float32) + bias

        iota = jax.lax.broadcasted_iota(jnp.int32, logits.shape, 1)
        cif = (63 - iota).astype(jnp.float32)
        work = logits
        w_cols = []
        i_cols = []
        for _ in range(_K):
            mx = jnp.max(work, axis=-1, keepdims=True)
            pk = jnp.max(jnp.where(work == mx, cif, -1.0), axis=-1,
                         keepdims=True)
            work = jnp.where(cif == pk, neg, work)
            w_cols.append(mx)
            i_cols.append(pk)

        lsel = jnp.concatenate(w_cols, axis=1)
        imat = (63.0 - jnp.concatenate(i_cols, axis=1)).astype(jnp.int32)
        m = w_cols[0]
        wexp = jnp.exp(lsel - m)
        w_ref[rows, :] = wexp * (1.0 / jnp.sum(wexp, axis=1, keepdims=True))
        id_ref[rows, :] = imat

        e = jnp.exp(logits - m)
        probs = e * (1.0 / jnp.sum(e, axis=-1, keepdims=True))
        selm = jnp.where(work == neg, 1.0, 0.0)
        psum_ref[...] += jnp.sum(probs, axis=0, keepdims=True)
        cnt_ref[...] += jnp.sum(selm, axis=0, keepdims=True)

    @pl.when(step == nsteps - 1)
    def _finish():
        scale = _ALPHA * _E / (float(_T) * _K * float(_T))
        aux = jnp.sum(psum_ref[...] * cnt_ref[...], keepdims=True) * scale
        aux_ref[...] = aux.reshape(1, 1)


@functools.partial(jax.jit, static_argnames=("interpret",))
def kernel(x, W, b, interpret=False):
    xt = x.reshape(_T, _D)
    wt = W.T
    b2 = b.reshape(1, _E)
    grid = (_T // _BT,)
    w_out, id_out, aux = pl.pallas_call(
        _router_block,
        grid=grid,
        in_specs=[
            pl.BlockSpec((_BT, _D), lambda i: (i, 0)),
            pl.BlockSpec((_D, _E), lambda i: (0, 0)),
            pl.BlockSpec((1, _E), lambda i: (0, 0)),
        ],
        out_specs=[
            pl.BlockSpec((_BT, _K), lambda i: (i, 0)),
            pl.BlockSpec((_BT, _K), lambda i: (i, 0)),
            pl.BlockSpec((1, 1), lambda i: (0, 0)),
        ],
        out_shape=[
            jax.ShapeDtypeStruct((_T, _K), jnp.float32),
            jax.ShapeDtypeStruct((_T, _K), jnp.int32),
            jax.ShapeDtypeStruct((1, 1), jnp.float32),
        ],
        scratch_shapes=[
            pltpu.VMEM((1, _E), jnp.float32),
            pltpu.VMEM((1, _E), jnp.float32),
        ],
        interpret=interpret,
    )(xt, wt, b2)
    return w_out, id_out, aux[0, 0]
